# baseline (speedup 1.0000x reference)
import functools

import jax
import jax.numpy as jnp
from jax import lax
from jax.experimental import pallas as pl
from jax.experimental.pallas import tpu as pltpu
from jax.experimental.pallas import tpu_sc as plsc

BATCH = 16384
CAT = 26
EMB = 32
CONT = 13
NC = 2
NS = 16
NW = NC * NS
BPW = BATCH // NW
QPW = BPW // 128
NCHUNK = CAT * QPW
CPAD = 16
WLEN = CAT * EMB + CPAD
NBUF = 8


def _body(cat, cont, table, wf, out, idx_v, cont_v, rows_v, acc_v, w_v, sem):
    wid = lax.axis_index("s") * NC + lax.axis_index("c")
    base = wid * BPW
    pltpu.sync_copy(cat.at[wid], idx_v)
    pltpu.sync_copy(cont.at[pl.ds(base, BPW)], cont_v)
    pltpu.sync_copy(wf, w_v)
    iota = lax.iota(jnp.int32, 16)

    wcb = [plsc.load_gather(w_v, [jnp.full((16,), CAT * EMB + d, jnp.int32)])
           for d in range(CONT)]
    bias = plsc.load_gather(w_v, [jnp.full((16,), CAT * EMB + CONT, jnp.int32)])

    def cont_group(g, carry):
        ridx = g * 16 + iota
        a = bias
        for d in range(CONT):
            v = plsc.load_gather(cont_v, [ridx, jnp.full((16,), d, jnp.int32)])
            a = a + v * wcb[d]
        acc_v[pl.ds(g * 16, 16)] = a
        return carry

    lax.fori_loop(0, BPW // 16, cont_group, 0)

    def fire(c, buf):
        pltpu.async_copy(table.at[idx_v.at[c]], rows_v.at[buf], sem.at[buf])

    for c0 in range(NBUF - 1):
        fire(c0, c0)

    def chunk(c, carry):
        buf = lax.rem(c, NBUF)

        @pl.when(c + NBUF - 1 < NCHUNK)
        def _():
            fire(c + NBUF - 1, lax.rem(c + NBUF - 1, NBUF))

        pltpu.make_async_copy(
            table.at[idx_v.at[c]], rows_v.at[buf], sem.at[buf]).wait()
        woff = (c // QPW) * EMB
        wb = [plsc.load_gather(w_v, [jnp.full((16,), d, jnp.int32) + woff])
              for d in range(EMB)]
        qbase = (c % QPW) * 128
        bidx = jnp.full((16,), buf, jnp.int32)

        def grp(g, inner_carry):
            ridx = g * 16 + iota
            ab = qbase + g * 16
            a0 = acc_v[pl.ds(ab, 16)]
            a1 = jnp.zeros((16,), jnp.float32)
            a2 = jnp.zeros((16,), jnp.float32)
            a3 = jnp.zeros((16,), jnp.float32)
            for d in range(0, EMB, 4):
                dd = [jnp.full((16,), d + k, jnp.int32) for k in range(4)]
                a0 = a0 + plsc.load_gather(rows_v, [bidx, ridx, dd[0]]) * wb[d]
                a1 = a1 + plsc.load_gather(rows_v, [bidx, ridx, dd[1]]) * wb[d + 1]
                a2 = a2 + plsc.load_gather(rows_v, [bidx, ridx, dd[2]]) * wb[d + 2]
                a3 = a3 + plsc.load_gather(rows_v, [bidx, ridx, dd[3]]) * wb[d + 3]
            acc_v[pl.ds(ab, 16)] = (a0 + a1) + (a2 + a3)
            return inner_carry

        lax.fori_loop(0, 8, grp, 0)
        return carry

    lax.fori_loop(0, NCHUNK, chunk, 0)

    def sig(g, carry):
        x = acc_v[pl.ds(g * 16, 16)]
        acc_v[pl.ds(g * 16, 16)] = 1.0 / (1.0 + jnp.exp(-x))
        return carry

    lax.fori_loop(0, BPW // 16, sig, 0)
    pltpu.sync_copy(acc_v, out.at[pl.ds(base, BPW)])


@functools.partial(jax.jit)
def _run(cat_idx, cont_p, table, wf):
    mesh = plsc.VectorSubcoreMesh(core_axis_name="c", subcore_axis_name="s")
    f = pl.kernel(
        _body,
        mesh=mesh,
        compiler_params=pltpu.CompilerParams(
            needs_layout_passes=False, use_tc_tiling_on_sc=False),
        out_type=jax.ShapeDtypeStruct((BATCH,), jnp.float32),
        scratch_types=[
            pltpu.VMEM((NCHUNK, 128), jnp.int32),
            pltpu.VMEM((BPW, CONT), jnp.float32),
            pltpu.VMEM((NBUF, 128, EMB), jnp.float32),
            pltpu.VMEM((BPW,), jnp.float32),
            pltpu.VMEM((WLEN,), jnp.float32),
            pltpu.SemaphoreType.DMA((NBUF,)),
        ],
    )
    return f(cat_idx, cont_p, table, wf)


def kernel(categorical_features, continous_features, emb_table, W, b):
    wf = jnp.concatenate([W[:, 0], b, jnp.zeros((2,), jnp.float32)])
    cat_fm = (categorical_features.astype(jnp.int32)
              .reshape(NW, QPW, 128, CAT)
              .transpose(0, 3, 1, 2)
              .reshape(NW, NCHUNK, 128))
    out = _run(cat_fm, continous_features, emb_table, wf)
    return out.reshape(BATCH, 1)


# speedup vs baseline: 1.1321x; 1.1321x over previous
import functools

import jax
import jax.numpy as jnp
from jax import lax
from jax.experimental import pallas as pl
from jax.experimental.pallas import tpu as pltpu
from jax.experimental.pallas import tpu_sc as plsc

BATCH = 16384
CAT = 26
EMB = 32
CONT = 13
NC = 2
NS = 16
NW = NC * NS
BPW = BATCH // NW
QPW = BPW // 128
NCHUNK = CAT * QPW
CPAD = 16
WLEN = CAT * EMB + CPAD
NBUF = 4
RPITCH = 34


def _body(cat, cont, table, wf, out,
          idx_v, cont_v, rows_v, acc_v, acc32_v, w_v, sem):
    wid = lax.axis_index("s") * NC + lax.axis_index("c")
    base = wid * BPW
    pltpu.sync_copy(cat.at[wid], idx_v)
    pltpu.sync_copy(cont.at[pl.ds(base, BPW)], cont_v)
    pltpu.sync_copy(wf, w_v)
    iota = lax.iota(jnp.int32, 16)

    wcb = [plsc.load_gather(w_v, [jnp.full((16,), CAT * EMB + d, jnp.int32)])
           for d in range(CONT)]
    bias = plsc.load_gather(w_v, [jnp.full((16,), CAT * EMB + CONT, jnp.int32)])

    def cont_group(g, carry):
        ridx = g * 16 + iota
        a = bias
        for d in range(CONT):
            v = plsc.load_gather(cont_v, [ridx, jnp.full((16,), d, jnp.int32)])
            a = a + v * wcb[d]
        acc_v[pl.ds(g * 16, 16)] = a
        return carry

    lax.fori_loop(0, BPW // 16, cont_group, 0)

    def fire(c, buf):
        pltpu.async_copy(table.at[idx_v.at[c]], rows_v.at[buf], sem.at[buf])

    for c0 in range(NBUF - 1):
        fire(c0, c0)

    zero16 = jnp.zeros((16,), jnp.float32)

    def z(r, carry):
        acc32_v[r, pl.ds(0, 16)] = zero16
        acc32_v[r, pl.ds(16, 16)] = zero16
        return carry

    lax.fori_loop(0, BPW, z, 0)

    def chunk(c, carry):
        buf = lax.rem(c, NBUF)

        @pl.when(c + NBUF - 1 < NCHUNK)
        def _():
            fire(c + NBUF - 1, lax.rem(c + NBUF - 1, NBUF))

        pltpu.make_async_copy(
            table.at[idx_v.at[c]], rows_v.at[buf], sem.at[buf]).wait()
        woff = (c // QPW) * EMB
        wb0 = w_v[pl.ds(woff, 16)]
        wb1 = w_v[pl.ds(woff + 16, 16)]
        qbase = (c % QPW) * 128

        def rowf(r4, inner_carry):
            for k in range(4):
                r = r4 * 4 + k
                a0 = acc32_v[qbase + r, pl.ds(0, 16)]
                a1 = acc32_v[qbase + r, pl.ds(16, 16)]
                v0 = rows_v[buf, r, pl.ds(0, 16)]
                v1 = rows_v[buf, r, pl.ds(16, 16)]
                acc32_v[qbase + r, pl.ds(0, 16)] = a0 + v0 * wb0
                acc32_v[qbase + r, pl.ds(16, 16)] = a1 + v1 * wb1
            return inner_carry

        lax.fori_loop(0, 32, rowf, 0)
        return carry

    lax.fori_loop(0, NCHUNK, chunk, 0)

    def sig(g, carry):
        ridx = g * 16 + iota
        x = acc_v[pl.ds(g * 16, 16)]
        for d in range(EMB):
            x = x + plsc.load_gather(acc32_v,
                                     [ridx, jnp.full((16,), d, jnp.int32)])
        acc_v[pl.ds(g * 16, 16)] = 1.0 / (1.0 + jnp.exp(-x))
        return carry

    lax.fori_loop(0, BPW // 16, sig, 0)
    pltpu.sync_copy(acc_v, out.at[pl.ds(base, BPW)])


@functools.partial(jax.jit)
def _run(cat_idx, cont_p, table, wf):
    mesh = plsc.VectorSubcoreMesh(core_axis_name="c", subcore_axis_name="s")
    f = pl.kernel(
        _body,
        mesh=mesh,
        compiler_params=pltpu.CompilerParams(
            needs_layout_passes=False, use_tc_tiling_on_sc=False),
        out_type=jax.ShapeDtypeStruct((BATCH,), jnp.float32),
        scratch_types=[
            pltpu.VMEM((NCHUNK, 128), jnp.int32),
            pltpu.VMEM((BPW, CONT), jnp.float32),
            pltpu.VMEM((NBUF, 128, EMB), jnp.float32),
            pltpu.VMEM((BPW,), jnp.float32),
            pltpu.VMEM((BPW, RPITCH), jnp.float32),
            pltpu.VMEM((WLEN,), jnp.float32),
            pltpu.SemaphoreType.DMA((NBUF,)),
        ],
    )
    return f(cat_idx, cont_p, table, wf)


def kernel(categorical_features, continous_features, emb_table, W, b):
    wf = jnp.concatenate([W[:, 0], b, jnp.zeros((2,), jnp.float32)])
    cat_fm = (categorical_features.astype(jnp.int32)
              .reshape(NW, QPW, 128, CAT)
              .transpose(0, 3, 1, 2)
              .reshape(NW, NCHUNK, 128))
    out = _run(cat_fm, continous_features, emb_table, wf)
    return out.reshape(BATCH, 1)


# speedup vs baseline: 1.2043x; 1.0638x over previous
import functools

import jax
import jax.numpy as jnp
from jax import lax
from jax.experimental import pallas as pl
from jax.experimental.pallas import tpu as pltpu
from jax.experimental.pallas import tpu_sc as plsc

BATCH = 16384
CAT = 26
EMB = 32
CONT = 13
NC = 2
NS = 16
NW = NC * NS
BPW = BATCH // NW
QPW = BPW // 128
NCHUNK = CAT * QPW
CPAD = 16
WLEN = CAT * EMB + CPAD
NBUF = 4
RPITCH = 34


def _body(cat, cont, table, wf, out,
          idx_v, cont_v, rows_v, acc_v, acc32_v, w_v, sem):
    wid = lax.axis_index("s") * NC + lax.axis_index("c")
    base = wid * BPW
    pltpu.sync_copy(cat.at[wid], idx_v)
    pltpu.sync_copy(cont.at[pl.ds(base, BPW)], cont_v)
    pltpu.sync_copy(wf, w_v)
    iota = lax.iota(jnp.int32, 16)

    wcb = [plsc.load_gather(w_v, [jnp.full((16,), CAT * EMB + d, jnp.int32)])
           for d in range(CONT)]
    bias = plsc.load_gather(w_v, [jnp.full((16,), CAT * EMB + CONT, jnp.int32)])

    def cont_group(g, carry):
        ridx = g * 16 + iota
        a = bias
        for d in range(CONT):
            v = plsc.load_gather(cont_v, [ridx, jnp.full((16,), d, jnp.int32)])
            a = a + v * wcb[d]
        acc_v[pl.ds(g * 16, 16)] = a
        return carry

    lax.fori_loop(0, BPW // 16, cont_group, 0)

    def fire(c, buf):
        pltpu.async_copy(table.at[idx_v.at[c]], rows_v.at[buf], sem.at[buf])

    def tc(t):
        return t // 2 + lax.rem(t, 2) * (NCHUNK // 2)

    for t0 in range(NBUF - 2):
        fire(t0 // 2 + (t0 % 2) * (NCHUNK // 2), t0)

    zero16 = jnp.zeros((16,), jnp.float32)

    def z(r, carry):
        acc32_v[r, pl.ds(0, 16)] = zero16
        acc32_v[r, pl.ds(16, 16)] = zero16
        return carry

    lax.fori_loop(0, BPW, z, 0)

    def pair(p, carry):
        for k in range(2):
            t = 2 * p + NBUF - 2 + k

            @pl.when(t < NCHUNK)
            def _():
                fire(tc(t), lax.rem(t, NBUF))

        buf1 = lax.rem(2 * p, NBUF)
        buf2 = lax.rem(2 * p + 1, NBUF)
        c1 = tc(2 * p)
        c2 = tc(2 * p + 1)
        pltpu.make_async_copy(
            table.at[idx_v.at[c1]], rows_v.at[buf1], sem.at[buf1]).wait()
        pltpu.make_async_copy(
            table.at[idx_v.at[c2]], rows_v.at[buf2], sem.at[buf2]).wait()
        woff1 = (p // QPW) * EMB
        woff2 = woff1 + (NCHUNK // 2 // QPW) * EMB
        wa0 = w_v[pl.ds(woff1, 16)]
        wa1 = w_v[pl.ds(woff1 + 16, 16)]
        wb0 = w_v[pl.ds(woff2, 16)]
        wb1 = w_v[pl.ds(woff2 + 16, 16)]
        qbase = lax.rem(p, QPW) * 128

        def rowf(r4, inner_carry):
            for k in range(4):
                r = r4 * 4 + k
                a0 = acc32_v[qbase + r, pl.ds(0, 16)]
                a1 = acc32_v[qbase + r, pl.ds(16, 16)]
                v0 = rows_v[buf1, r, pl.ds(0, 16)]
                v1 = rows_v[buf1, r, pl.ds(16, 16)]
                u0 = rows_v[buf2, r, pl.ds(0, 16)]
                u1 = rows_v[buf2, r, pl.ds(16, 16)]
                acc32_v[qbase + r, pl.ds(0, 16)] = (a0 + v0 * wa0) + u0 * wb0
                acc32_v[qbase + r, pl.ds(16, 16)] = (a1 + v1 * wa1) + u1 * wb1
            return inner_carry

        lax.fori_loop(0, 32, rowf, 0)
        return carry

    lax.fori_loop(0, NCHUNK // 2, pair, 0)

    def sig(g, carry):
        ridx = g * 16 + iota
        x = acc_v[pl.ds(g * 16, 16)]
        for d in range(EMB):
            x = x + plsc.load_gather(acc32_v,
                                     [ridx, jnp.full((16,), d, jnp.int32)])
        acc_v[pl.ds(g * 16, 16)] = 1.0 / (1.0 + jnp.exp(-x))
        return carry

    lax.fori_loop(0, BPW // 16, sig, 0)
    pltpu.sync_copy(acc_v, out.at[pl.ds(base, BPW)])


@functools.partial(jax.jit)
def _run(cat_idx, cont_p, table, wf):
    mesh = plsc.VectorSubcoreMesh(core_axis_name="c", subcore_axis_name="s")
    f = pl.kernel(
        _body,
        mesh=mesh,
        compiler_params=pltpu.CompilerParams(
            needs_layout_passes=False, use_tc_tiling_on_sc=False),
        out_type=jax.ShapeDtypeStruct((BATCH,), jnp.float32),
        scratch_types=[
            pltpu.VMEM((NCHUNK, 128), jnp.int32),
            pltpu.VMEM((BPW, CONT), jnp.float32),
            pltpu.VMEM((NBUF, 128, EMB), jnp.float32),
            pltpu.VMEM((BPW,), jnp.float32),
            pltpu.VMEM((BPW, RPITCH), jnp.float32),
            pltpu.VMEM((WLEN,), jnp.float32),
            pltpu.SemaphoreType.DMA((NBUF,)),
        ],
    )
    return f(cat_idx, cont_p, table, wf)


def kernel(categorical_features, continous_features, emb_table, W, b):
    wf = jnp.concatenate([W[:, 0], b, jnp.zeros((2,), jnp.float32)])
    cat_fm = (categorical_features.astype(jnp.int32)
              .reshape(NW, QPW, 128, CAT)
              .transpose(0, 3, 1, 2)
              .reshape(NW, NCHUNK, 128))
    out = _run(cat_fm, continous_features, emb_table, wf)
    return out.reshape(BATCH, 1)


# speedup vs baseline: 1.2101x; 1.0048x over previous
import functools

import jax
import jax.numpy as jnp
from jax import lax
from jax.experimental import pallas as pl
from jax.experimental.pallas import tpu as pltpu
from jax.experimental.pallas import tpu_sc as plsc

BATCH = 16384
CAT = 26
EMB = 32
CONT = 13
NC = 2
NS = 16
NW = NC * NS
BPW = BATCH // NW
QPW = BPW // 128
NCHUNK = CAT * QPW
CPAD = 16
WLEN = CAT * EMB + CPAD
NBUF = 6
RPITCH = 34


def _body(cat, cont, table, wf, out,
          idx_v, cont_v, rows_v, acc_v, acc32_v, w_v, sem):
    wid = lax.axis_index("s") * NC + lax.axis_index("c")
    base = wid * BPW
    pltpu.sync_copy(cat.at[wid], idx_v)
    pltpu.sync_copy(cont.at[pl.ds(base, BPW)], cont_v)
    pltpu.sync_copy(wf, w_v)
    iota = lax.iota(jnp.int32, 16)

    wcb = [plsc.load_gather(w_v, [jnp.full((16,), CAT * EMB + d, jnp.int32)])
           for d in range(CONT)]
    bias = plsc.load_gather(w_v, [jnp.full((16,), CAT * EMB + CONT, jnp.int32)])

    def cont_group(g, carry):
        ridx = g * 16 + iota
        a = bias
        for d in range(CONT):
            v = plsc.load_gather(cont_v, [ridx, jnp.full((16,), d, jnp.int32)])
            a = a + v * wcb[d]
        acc_v[pl.ds(g * 16, 16)] = a
        return carry

    lax.fori_loop(0, BPW // 16, cont_group, 0)

    def fire(c, buf):
        pltpu.async_copy(table.at[idx_v.at[c]], rows_v.at[buf], sem.at[buf])

    def tc(t):
        return t // 2 + lax.rem(t, 2) * (NCHUNK // 2)

    for t0 in range(NBUF - 2):
        fire(t0 // 2 + (t0 % 2) * (NCHUNK // 2), t0)

    zero16 = jnp.zeros((16,), jnp.float32)

    def z(r, carry):
        acc32_v[r, pl.ds(0, 16)] = zero16
        acc32_v[r, pl.ds(16, 16)] = zero16
        return carry

    lax.fori_loop(0, BPW, z, 0)

    def pair(p, carry):
        for k in range(2):
            t = 2 * p + NBUF - 2 + k

            @pl.when(t < NCHUNK)
            def _():
                fire(tc(t), lax.rem(t, NBUF))

        buf1 = lax.rem(2 * p, NBUF)
        buf2 = lax.rem(2 * p + 1, NBUF)
        c1 = tc(2 * p)
        c2 = tc(2 * p + 1)
        pltpu.make_async_copy(
            table.at[idx_v.at[c1]], rows_v.at[buf1], sem.at[buf1]).wait()
        pltpu.make_async_copy(
            table.at[idx_v.at[c2]], rows_v.at[buf2], sem.at[buf2]).wait()
        woff1 = (p // QPW) * EMB
        woff2 = woff1 + (NCHUNK // 2 // QPW) * EMB
        wa0 = w_v[pl.ds(woff1, 16)]
        wa1 = w_v[pl.ds(woff1 + 16, 16)]
        wb0 = w_v[pl.ds(woff2, 16)]
        wb1 = w_v[pl.ds(woff2 + 16, 16)]
        qbase = lax.rem(p, QPW) * 128

        def rowf(r4, inner_carry):
            for k in range(8):
                r = r4 * 8 + k
                a0 = acc32_v[qbase + r, pl.ds(0, 16)]
                a1 = acc32_v[qbase + r, pl.ds(16, 16)]
                v0 = rows_v[buf1, r, pl.ds(0, 16)]
                v1 = rows_v[buf1, r, pl.ds(16, 16)]
                u0 = rows_v[buf2, r, pl.ds(0, 16)]
                u1 = rows_v[buf2, r, pl.ds(16, 16)]
                acc32_v[qbase + r, pl.ds(0, 16)] = (a0 + v0 * wa0) + u0 * wb0
                acc32_v[qbase + r, pl.ds(16, 16)] = (a1 + v1 * wa1) + u1 * wb1
            return inner_carry

        lax.fori_loop(0, 16, rowf, 0)
        return carry

    lax.fori_loop(0, NCHUNK // 2, pair, 0)

    def sig(g, carry):
        ridx = g * 16 + iota
        x = acc_v[pl.ds(g * 16, 16)]
        for d in range(EMB):
            x = x + plsc.load_gather(acc32_v,
                                     [ridx, jnp.full((16,), d, jnp.int32)])
        acc_v[pl.ds(g * 16, 16)] = 1.0 / (1.0 + jnp.exp(-x))
        return carry

    lax.fori_loop(0, BPW // 16, sig, 0)
    pltpu.sync_copy(acc_v, out.at[pl.ds(base, BPW)])


@functools.partial(jax.jit)
def _run(cat_idx, cont_p, table, wf):
    mesh = plsc.VectorSubcoreMesh(core_axis_name="c", subcore_axis_name="s")
    f = pl.kernel(
        _body,
        mesh=mesh,
        compiler_params=pltpu.CompilerParams(
            needs_layout_passes=False, use_tc_tiling_on_sc=False),
        out_type=jax.ShapeDtypeStruct((BATCH,), jnp.float32),
        scratch_types=[
            pltpu.VMEM((NCHUNK, 128), jnp.int32),
            pltpu.VMEM((BPW, CONT), jnp.float32),
            pltpu.VMEM((NBUF, 128, EMB), jnp.float32),
            pltpu.VMEM((BPW,), jnp.float32),
            pltpu.VMEM((BPW, RPITCH), jnp.float32),
            pltpu.VMEM((WLEN,), jnp.float32),
            pltpu.SemaphoreType.DMA((NBUF,)),
        ],
    )
    return f(cat_idx, cont_p, table, wf)


def kernel(categorical_features, continous_features, emb_table, W, b):
    wf = jnp.concatenate([W[:, 0], b, jnp.zeros((2,), jnp.float32)])
    cat_fm = (categorical_features.astype(jnp.int32)
              .reshape(NW, QPW, 128, CAT)
              .transpose(0, 3, 1, 2)
              .reshape(NW, NCHUNK, 128))
    out = _run(cat_fm, continous_features, emb_table, wf)
    return out.reshape(BATCH, 1)


# speedup vs baseline: 1.2138x; 1.0031x over previous
import functools

import jax
import jax.numpy as jnp
from jax import lax
from jax.experimental import pallas as pl
from jax.experimental.pallas import tpu as pltpu
from jax.experimental.pallas import tpu_sc as plsc

BATCH = 16384
CAT = 26
EMB = 32
CONT = 13
NC = 2
NS = 16
NW = NC * NS
BPW = BATCH // NW
QPW = BPW // 128
NCHUNK = CAT * QPW
CPAD = 16
WLEN = CAT * EMB + CPAD
NBUF = 6
RPITCH = 34


def _body(cat, cont, table, wf, out,
          idx_v, cont_v, rows_v, acc_v, acc32_v, w_v, sem):
    wid = lax.axis_index("s") * NC + lax.axis_index("c")
    base = wid * BPW
    pltpu.sync_copy(cat.at[wid], idx_v)
    pltpu.sync_copy(cont.at[pl.ds(base, BPW)], cont_v)
    pltpu.sync_copy(wf, w_v)
    iota = lax.iota(jnp.int32, 16)

    def fire(c, buf):
        pltpu.async_copy(table.at[idx_v.at[c]], rows_v.at[buf], sem.at[buf])

    def tc(t):
        return t // 2 + lax.rem(t, 2) * (NCHUNK // 2)

    for t0 in range(NBUF - 2):
        fire(t0 // 2 + (t0 % 2) * (NCHUNK // 2), t0)

    zero16 = jnp.zeros((16,), jnp.float32)
    wcv = w_v[pl.ds(CAT * EMB, 16)]

    def z(r, carry):
        acc32_v[r, pl.ds(0, 16)] = cont_v[r, pl.ds(0, 16)] * wcv
        acc32_v[r, pl.ds(16, 16)] = zero16
        return carry

    lax.fori_loop(0, BPW, z, 0)

    def pair(p, carry):
        for k in range(2):
            t = 2 * p + NBUF - 2 + k

            @pl.when(t < NCHUNK)
            def _():
                fire(tc(t), lax.rem(t, NBUF))

        buf1 = lax.rem(2 * p, NBUF)
        buf2 = lax.rem(2 * p + 1, NBUF)
        c1 = tc(2 * p)
        c2 = tc(2 * p + 1)
        pltpu.make_async_copy(
            table.at[idx_v.at[c1]], rows_v.at[buf1], sem.at[buf1]).wait()
        pltpu.make_async_copy(
            table.at[idx_v.at[c2]], rows_v.at[buf2], sem.at[buf2]).wait()
        woff1 = (p // QPW) * EMB
        woff2 = woff1 + (NCHUNK // 2 // QPW) * EMB
        wa0 = w_v[pl.ds(woff1, 16)]
        wa1 = w_v[pl.ds(woff1 + 16, 16)]
        wb0 = w_v[pl.ds(woff2, 16)]
        wb1 = w_v[pl.ds(woff2 + 16, 16)]
        qbase = lax.rem(p, QPW) * 128

        def rowf(r4, inner_carry):
            for k in range(8):
                r = r4 * 8 + k
                a0 = acc32_v[qbase + r, pl.ds(0, 16)]
                a1 = acc32_v[qbase + r, pl.ds(16, 16)]
                v0 = rows_v[buf1, r, pl.ds(0, 16)]
                v1 = rows_v[buf1, r, pl.ds(16, 16)]
                u0 = rows_v[buf2, r, pl.ds(0, 16)]
                u1 = rows_v[buf2, r, pl.ds(16, 16)]
                acc32_v[qbase + r, pl.ds(0, 16)] = (a0 + v0 * wa0) + u0 * wb0
                acc32_v[qbase + r, pl.ds(16, 16)] = (a1 + v1 * wa1) + u1 * wb1
            return inner_carry

        lax.fori_loop(0, 16, rowf, 0)
        return carry

    lax.fori_loop(0, NCHUNK // 2, pair, 0)

    def sig(g, carry):
        ridx = g * 16 + iota
        x = jnp.zeros((16,), jnp.float32)
        for d in range(EMB):
            x = x + plsc.load_gather(acc32_v,
                                     [ridx, jnp.full((16,), d, jnp.int32)])
        acc_v[pl.ds(g * 16, 16)] = 1.0 / (1.0 + jnp.exp(-x))
        return carry

    lax.fori_loop(0, BPW // 16, sig, 0)
    pltpu.sync_copy(acc_v, out.at[pl.ds(base, BPW)])


@functools.partial(jax.jit)
def _run(cat_idx, cont_p, table, wf):
    mesh = plsc.VectorSubcoreMesh(core_axis_name="c", subcore_axis_name="s")
    f = pl.kernel(
        _body,
        mesh=mesh,
        compiler_params=pltpu.CompilerParams(
            needs_layout_passes=False, use_tc_tiling_on_sc=False),
        out_type=jax.ShapeDtypeStruct((BATCH,), jnp.float32),
        scratch_types=[
            pltpu.VMEM((NCHUNK, 128), jnp.int32),
            pltpu.VMEM((BPW, CPAD), jnp.float32),
            pltpu.VMEM((NBUF, 128, EMB), jnp.float32),
            pltpu.VMEM((BPW,), jnp.float32),
            pltpu.VMEM((BPW, RPITCH), jnp.float32),
            pltpu.VMEM((WLEN,), jnp.float32),
            pltpu.SemaphoreType.DMA((NBUF,)),
        ],
    )
    return f(cat_idx, cont_p, table, wf)


def kernel(categorical_features, continous_features, emb_table, W, b):
    wf = jnp.concatenate([W[:, 0], b, jnp.zeros((2,), jnp.float32)])
    cat_fm = (categorical_features.astype(jnp.int32)
              .reshape(NW, QPW, 128, CAT)
              .transpose(0, 3, 1, 2)
              .reshape(NW, NCHUNK, 128))
    cont16 = jnp.concatenate(
        [continous_features,
         jnp.ones((BATCH, 1), jnp.float32),
         jnp.zeros((BATCH, 2), jnp.float32)], axis=1)
    out = _run(cat_fm, cont16, emb_table, wf)
    return out.reshape(BATCH, 1)
